# baseline (speedup 1.0000x reference)
import jax
import jax.numpy as jnp
import numpy as np
from jax import lax
from jax.experimental import pallas as pl
from jax.experimental.pallas import tpu as pltpu
from jax.experimental.pallas import tpu_sc as plsc

DIMH = 128
BATCH = 512
NC, NS, L = 2, 16, 16
NW = NC * NS
RPW = BATCH // NW
NREL = 3
NOUT = 8

_SLOT_SRC = [
    (0, 0), (0, 1),
    (1, 0), (1, 1), (1, 2),
    (2, 0), (2, 2),
    (3, 1), (3, 2),
    (4, 0), (4, 1),
    (5, 0), (5, 2),
    (2, 1), (3, 0), (5, 1),
]
_NA = 5
_NB = 8


def _vsqrt(x):
    xc = jnp.maximum(x, jnp.float32(1e-30))
    i = lax.bitcast_convert_type(xc, jnp.int32)
    i = jnp.int32(0x5F3759DF) - jnp.right_shift(i, jnp.int32(1))
    g = lax.bitcast_convert_type(i, jnp.float32)
    for _ in range(3):
        g = g * (jnp.float32(1.5) - jnp.float32(0.5) * xc * g * g)
    return x * g


def _relu(x):
    return jnp.maximum(x, jnp.float32(0.0))


def _sc_body(idx_hbm, cls_hbm, rel_hbm, out_hbm, *refs):
    idx_v, bufa, bufb, bufr, partials, sema, semb = refs

    wid = lax.axis_index("s") * NC + lax.axis_index("c")

    nidx = len(_SLOT_SRC) * RPW
    pltpu.sync_copy(idx_hbm.at[pl.ds(wid * nidx, nidx)], idx_v)

    iota = lax.iota(jnp.int32, L)

    c1 = pltpu.async_copy(cls_hbm.at[idx_v.at[pl.ds(0, _NA * RPW)]], bufa,
                          sema)
    c2 = pltpu.async_copy(cls_hbm.at[idx_v.at[pl.ds(_NA * RPW, _NB * RPW)]],
                          bufb, semb)
    c3 = pltpu.async_copy(
        rel_hbm.at[idx_v.at[pl.ds((_NA + _NB) * RPW, NREL * RPW)]], bufr, semb)

    zero = jnp.zeros((L,), jnp.float32)
    half = jnp.float32(0.5)

    row_a = [iota + jnp.int32(k * RPW) for k in range(_NA)]
    row_b = [iota + jnp.int32(k * RPW) for k in range(_NB)]
    row_r = [iota + jnp.int32(k * RPW) for k in range(NREL)]

    def upk(w):
        wb = plsc.bitcast(w, jnp.bfloat16)
        return plsc.unpack(wb, format=plsc.PackFormat.INTERLEAVED)

    def alo(col, dsp):
        return upk(plsc.load_gather(bufa, [row_a[col], dsp]))

    def ahi(col, dsph):
        a, b = upk(plsc.load_gather(bufa, [row_a[col], dsph]))
        return jnp.abs(a), jnp.abs(b)

    def blo(col, dsp):
        return upk(plsc.load_gather(bufb, [row_b[col], dsp]))

    def bhi(col, dsph):
        a, b = upk(plsc.load_gather(bufb, [row_b[col], dsph]))
        return jnp.abs(a), jnp.abs(b)

    def rlo(k, dsp):
        return upk(plsc.load_gather(bufr, [row_r[k], dsp]))

    NPAIR = DIMH // 2

    def rot(dd):
        dsp = jnp.bitwise_and(iota + dd, jnp.int32(NPAIR - 1))
        return dsp, dsp + jnp.int32(NPAIR)

    c1.wait()

    @plsc.parallel_loop(0, NPAIR, step=1, unroll=1, carry=(zero, zero, zero))
    def loop_a(dd, carry):
        s1, ar, br = carry
        dsp, dsph = rot(dd)
        a0 = alo(0, dsp)
        a1 = alo(1, dsp)
        h0 = ahi(0, dsph)
        h1 = ahi(1, dsph)
        c2v = ahi(2, dsph)
        c1v = alo(2, dsp)
        d1 = alo(3, dsp)
        d2 = ahi(3, dsph)
        e1 = alo(4, dsp)
        e2 = ahi(4, dsph)
        for u in range(2):
            t = _relu(jnp.abs(a0[u] - a1[u]) + h0[u] - h1[u])
            s1 = s1 + t * t
            st = jnp.maximum(c1v[u] - c2v[u], d1[u] - d2[u])
            en = jnp.minimum(c1v[u] + c2v[u], d1[u] + d2[u])
            diff = st - en
            ta = _relu(jnp.abs(half * (st + en) - e1[u])
                       + half * jnp.abs(diff) - e2[u])
            ar = ar + ta * ta
            tb = _relu(diff)
            br = br + tb * tb
        return s1, ar, br

    s134, a2, b2 = loop_a

    c2.wait()
    c3.wait()

    @plsc.parallel_loop(0, NPAIR, step=1, unroll=1, carry=(zero, zero, zero))
    def loop_b(dd, carry):
        s34, djr, negr = carry
        dsp, dsph = rot(dd)
        b0 = blo(0, dsp)
        b1 = blo(1, dsp)
        b2v = blo(2, dsp)
        b3 = blo(3, dsp)
        b4 = blo(4, dsp)
        b5 = blo(5, dsp)
        b6 = blo(6, dsp)
        b7 = blo(7, dsp)
        g0 = bhi(0, dsph)
        g1 = bhi(1, dsph)
        g2 = bhi(2, dsph)
        g3 = bhi(3, dsph)
        g4 = bhi(4, dsph)
        g5 = bhi(5, dsph)
        g6 = bhi(6, dsph)
        g7 = bhi(7, dsph)
        r0 = rlo(0, dsp)
        r1 = rlo(1, dsp)
        r2 = rlo(2, dsp)
        for u in range(2):
            t = _relu(jnp.abs(b0[u] + r0[u] - b1[u]) + g0[u] - g1[u])
            s34 = s34 + t * t
            t = _relu(jnp.abs(b2v[u] - r1[u] - b3[u]) - g2[u] - g3[u])
            s34 = s34 + t * t
            t = _relu(jnp.abs(b4[u] - b5[u]) - g4[u] - g5[u])
            djr = djr + t * t
            t = _relu(jnp.abs(b6[u] + r2[u] - b7[u]) - g6[u] - g7[u])
            negr = negr + t * t
        return s34, djr, negr

    s34, djr, negr = loop_b
    s134 = s134 + s34

    two = jnp.float32(2.0)
    djv = _relu(two - _vsqrt(djr))
    negv = two - _vsqrt(negr)

    partials[0, :] = s134
    partials[1, :] = a2
    partials[2, :] = _vsqrt(a2)
    partials[3, :] = b2
    partials[4, :] = _vsqrt(b2)
    partials[5, :] = djv * djv
    partials[6, :] = negv * negv
    partials[7, :] = zero
    pltpu.sync_copy(partials, out_hbm.at[wid])


def _finish_body(x_ref, o_ref):
    x = x_ref[...]
    inv = jnp.float32(1.0 / BATCH)
    s134 = jnp.sum(x[:, 0, :])
    sa2 = jnp.sum(x[:, 1, :])
    sa = jnp.sum(x[:, 2, :])
    sb2 = jnp.sum(x[:, 3, :])
    sb = jnp.sum(x[:, 4, :])
    sdj = jnp.sum(x[:, 5, :])
    sneg = jnp.sum(x[:, 6, :])
    loss2 = inv * sa2 + inv * sb2 + jnp.float32(2.0) * (inv * sa) * (inv * sb)
    total = inv * s134 + loss2 + inv * sdj + inv * sneg
    o_ref[...] = jnp.broadcast_to(total, (1, 1))


@jax.jit
def _run(idx3, classEmb, relEmb):
    mesh = plsc.VectorSubcoreMesh(core_axis_name="c", subcore_axis_name="s")
    scratch = [
        pltpu.VMEM((len(_SLOT_SRC) * RPW,), jnp.int32),
        pltpu.VMEM((_NA * RPW, DIMH), jnp.int32),
        pltpu.VMEM((_NB * RPW, DIMH), jnp.int32),
        pltpu.VMEM((NREL * RPW, DIMH), jnp.int32),
        pltpu.VMEM((NOUT, L), jnp.float32),
        pltpu.SemaphoreType.DMA,
        pltpu.SemaphoreType.DMA,
    ]
    sc_call = pl.kernel(
        _sc_body,
        out_type=jax.ShapeDtypeStruct((NW, NOUT, L), jnp.float32),
        mesh=mesh,
        scratch_types=scratch,
        compiler_params=pltpu.CompilerParams(needs_layout_passes=False),
    )
    partials = sc_call(idx3, classEmb, relEmb)
    out = pl.pallas_call(
        _finish_body,
        out_shape=jax.ShapeDtypeStruct((1, 1), jnp.float32),
    )(partials)
    return jnp.reshape(out, ())


def _build_perm():
    sizes = [2048 * n for n in (2, 3, 3, 3, 2, 3)]
    offs = np.cumsum([0] + sizes)[:-1]
    ncols = (2, 3, 3, 3, 2, 3)
    perm = np.empty((NW, len(_SLOT_SRC), RPW), np.int32)
    for w in range(NW):
        for slot, (s, c) in enumerate(_SLOT_SRC):
            for i in range(RPW):
                perm[w, slot, i] = offs[s] + (w * RPW + i) * ncols[s] + c
    return perm.reshape(-1)


_PERM = _build_perm()


def kernel(nf1, nf2, nf3, nf4, disjoint, nf3_neg, classEmb, relEmb):
    b = BATCH
    i32 = jnp.int32
    cols = [
        nf1[:b, 0], nf1[:b, 1],
        nf2[:b, 0], nf2[:b, 1], nf2[:b, 2],
        nf3[:b, 0], nf3[:b, 2],
        nf4[:b, 1], nf4[:b, 2],
        disjoint[:b, 0], disjoint[:b, 1],
        nf3_neg[:b, 0], nf3_neg[:b, 2],
        nf3[:b, 1], nf4[:b, 0], nf3_neg[:b, 1],
    ]
    idx_all = jnp.stack([c.astype(i32) for c in cols], axis=0)
    idx3 = idx_all.reshape(16, NW, RPW).transpose(1, 0, 2).reshape(NW * 256)
    cls_p = lax.bitcast_convert_type(
        classEmb.astype(jnp.bfloat16).reshape(-1, DIMH, 2), jnp.int32)
    rel_p = lax.bitcast_convert_type(
        relEmb.astype(jnp.bfloat16).reshape(-1, DIMH // 2, 2), jnp.int32)
    rel_p = jnp.pad(rel_p, ((0, 0), (0, DIMH // 2)))
    return _run(idx3, cls_p, rel_p)


# speedup vs baseline: 1.1542x; 1.1542x over previous
import jax
import jax.numpy as jnp
import numpy as np
from jax import lax
from jax.experimental import pallas as pl
from jax.experimental.pallas import tpu as pltpu
from jax.experimental.pallas import tpu_sc as plsc

DIMH = 128
BATCH = 512
NC, NS, L = 2, 16, 16
NW = NC * NS
RPW = BATCH // NW
NREL = 3
NOUT = 8

_SLOT_SRC = [
    (0, 0), (0, 1),
    (1, 0), (1, 1), (1, 2),
    (2, 0), (2, 2),
    (3, 1), (3, 2),
    (4, 0), (4, 1),
    (5, 0), (5, 2),
    (2, 1), (3, 0), (5, 1),
]
_NA = 5
_NB = 8


def _vsqrt(x):
    xc = jnp.maximum(x, jnp.float32(1e-30))
    i = lax.bitcast_convert_type(xc, jnp.int32)
    i = jnp.int32(0x5F3759DF) - jnp.right_shift(i, jnp.int32(1))
    g = lax.bitcast_convert_type(i, jnp.float32)
    for _ in range(3):
        g = g * (jnp.float32(1.5) - jnp.float32(0.5) * xc * g * g)
    return x * g


def _relu(x):
    return jnp.maximum(x, jnp.float32(0.0))


def _sc_body(idx_hbm, cls_hbm, rel_hbm, out_hbm, *refs):
    idx_v, bufa, bufb, bufr, partials, sema, semb = refs

    wid = lax.axis_index("s") * NC + lax.axis_index("c")

    nidx = len(_SLOT_SRC) * RPW
    pltpu.sync_copy(idx_hbm.at[pl.ds(wid * nidx, nidx)], idx_v)

    iota = lax.iota(jnp.int32, L)

    c1 = pltpu.async_copy(cls_hbm.at[idx_v.at[pl.ds(0, _NA * RPW)]], bufa,
                          sema)
    c2 = pltpu.async_copy(cls_hbm.at[idx_v.at[pl.ds(_NA * RPW, _NB * RPW)]],
                          bufb, semb)
    c3 = pltpu.async_copy(
        rel_hbm.at[idx_v.at[pl.ds((_NA + _NB) * RPW, NREL * RPW)]], bufr, semb)

    zero = jnp.zeros((L,), jnp.float32)
    half = jnp.float32(0.5)

    row_a = [iota + jnp.int32(k * RPW) for k in range(_NA)]
    row_b = [iota + jnp.int32(k * RPW) for k in range(_NB)]
    row_r = [iota + jnp.int32(k * RPW) for k in range(NREL)]

    def alo(col, dsp):
        return plsc.load_gather(bufa, [row_a[col], dsp])

    def ahi(col, dsph):
        return jnp.abs(plsc.load_gather(bufa, [row_a[col], dsph]))

    def blo(col, dsp):
        return plsc.load_gather(bufb, [row_b[col], dsp])

    def bhi(col, dsph):
        return jnp.abs(plsc.load_gather(bufb, [row_b[col], dsph]))

    def rlo(k, dsp):
        return plsc.load_gather(bufr, [row_r[k], dsp])

    def rot(dd):
        dsp = jnp.bitwise_and(iota + dd, jnp.int32(DIMH - 1))
        return dsp, dsp + jnp.int32(DIMH)

    c1.wait()

    @plsc.parallel_loop(0, DIMH, step=1, unroll=1, carry=(zero, zero, zero))
    def loop_a(dd, carry):
        s1, ar, br = carry
        dsp, dsph = rot(dd)
        t = _relu(jnp.abs(alo(0, dsp) - alo(1, dsp))
                  + ahi(0, dsph) - ahi(1, dsph))
        s1 = s1 + t * t
        c1v = alo(2, dsp)
        c2v = ahi(2, dsph)
        d1 = alo(3, dsp)
        d2 = ahi(3, dsph)
        e1 = alo(4, dsp)
        e2 = ahi(4, dsph)
        st = jnp.maximum(c1v - c2v, d1 - d2)
        en = jnp.minimum(c1v + c2v, d1 + d2)
        diff = st - en
        ta = _relu(jnp.abs(half * (st + en) - e1) + half * jnp.abs(diff) - e2)
        ar = ar + ta * ta
        tb = _relu(diff)
        br = br + tb * tb
        return s1, ar, br

    s134, a2, b2 = loop_a

    c2.wait()
    c3.wait()

    @plsc.parallel_loop(0, DIMH, step=1, unroll=1, carry=(zero, zero, zero))
    def loop_b(dd, carry):
        s34, djr, negr = carry
        dsp, dsph = rot(dd)
        t = _relu(jnp.abs(blo(0, dsp) + rlo(0, dsp) - blo(1, dsp))
                  + bhi(0, dsph) - bhi(1, dsph))
        s34 = s34 + t * t
        t = _relu(jnp.abs(blo(2, dsp) - rlo(1, dsp) - blo(3, dsp))
                  - bhi(2, dsph) - bhi(3, dsph))
        s34 = s34 + t * t
        t = _relu(jnp.abs(blo(4, dsp) - blo(5, dsp))
                  - bhi(4, dsph) - bhi(5, dsph))
        djr = djr + t * t
        t = _relu(jnp.abs(blo(6, dsp) + rlo(2, dsp) - blo(7, dsp))
                  - bhi(6, dsph) - bhi(7, dsph))
        negr = negr + t * t
        return s34, djr, negr

    s34, djr, negr = loop_b
    s134 = s134 + s34

    two = jnp.float32(2.0)
    djv = _relu(two - _vsqrt(djr))
    negv = two - _vsqrt(negr)

    partials[0, :] = s134
    partials[1, :] = a2
    partials[2, :] = _vsqrt(a2)
    partials[3, :] = b2
    partials[4, :] = _vsqrt(b2)
    partials[5, :] = djv * djv
    partials[6, :] = negv * negv
    partials[7, :] = zero
    pltpu.sync_copy(partials, out_hbm.at[wid])


def _finish_body(x_ref, o_ref):
    x = x_ref[...]
    inv = jnp.float32(1.0 / BATCH)
    s134 = jnp.sum(x[:, 0, :])
    sa2 = jnp.sum(x[:, 1, :])
    sa = jnp.sum(x[:, 2, :])
    sb2 = jnp.sum(x[:, 3, :])
    sb = jnp.sum(x[:, 4, :])
    sdj = jnp.sum(x[:, 5, :])
    sneg = jnp.sum(x[:, 6, :])
    loss2 = inv * sa2 + inv * sb2 + jnp.float32(2.0) * (inv * sa) * (inv * sb)
    total = inv * s134 + loss2 + inv * sdj + inv * sneg
    o_ref[...] = jnp.broadcast_to(total, (1, 1))


@jax.jit
def _run(idx3, classEmb, relEmb):
    mesh = plsc.VectorSubcoreMesh(core_axis_name="c", subcore_axis_name="s")
    scratch = [
        pltpu.VMEM((len(_SLOT_SRC) * RPW,), jnp.int32),
        pltpu.VMEM((_NA * RPW, 2 * DIMH), jnp.float32),
        pltpu.VMEM((_NB * RPW, 2 * DIMH), jnp.float32),
        pltpu.VMEM((NREL * RPW, DIMH), jnp.float32),
        pltpu.VMEM((NOUT, L), jnp.float32),
        pltpu.SemaphoreType.DMA,
        pltpu.SemaphoreType.DMA,
    ]
    sc_call = pl.kernel(
        _sc_body,
        out_type=jax.ShapeDtypeStruct((NW, NOUT, L), jnp.float32),
        mesh=mesh,
        scratch_types=scratch,
        compiler_params=pltpu.CompilerParams(needs_layout_passes=False),
    )
    partials = sc_call(idx3, classEmb, relEmb)
    out = pl.pallas_call(
        _finish_body,
        out_shape=jax.ShapeDtypeStruct((1, 1), jnp.float32),
    )(partials)
    return jnp.reshape(out, ())


def _build_perm():
    sizes = [2048 * n for n in (2, 3, 3, 3, 2, 3)]
    offs = np.cumsum([0] + sizes)[:-1]
    ncols = (2, 3, 3, 3, 2, 3)
    perm = np.empty((NW, len(_SLOT_SRC), RPW), np.int32)
    for w in range(NW):
        for slot, (s, c) in enumerate(_SLOT_SRC):
            for i in range(RPW):
                perm[w, slot, i] = offs[s] + (w * RPW + i) * ncols[s] + c
    return perm.reshape(-1)


_PERM = _build_perm()


def kernel(nf1, nf2, nf3, nf4, disjoint, nf3_neg, classEmb, relEmb):
    b = BATCH
    i32 = jnp.int32
    cols = [
        nf1[:b, 0], nf1[:b, 1],
        nf2[:b, 0], nf2[:b, 1], nf2[:b, 2],
        nf3[:b, 0], nf3[:b, 2],
        nf4[:b, 1], nf4[:b, 2],
        disjoint[:b, 0], disjoint[:b, 1],
        nf3_neg[:b, 0], nf3_neg[:b, 2],
        nf3[:b, 1], nf4[:b, 0], nf3_neg[:b, 1],
    ]
    idx_all = jnp.stack([c.astype(i32) for c in cols], axis=0)
    idx3 = idx_all.reshape(16, NW, RPW).transpose(1, 0, 2).reshape(NW * 256)
    return _run(idx3, classEmb, relEmb)


# speedup vs baseline: 1.1599x; 1.0049x over previous
import jax
import jax.numpy as jnp
from jax import lax
from jax.experimental import pallas as pl
from jax.experimental.pallas import tpu as pltpu
from jax.experimental.pallas import tpu_sc as plsc

DIMH = 128
BATCH = 512
NC, NS, L = 2, 16, 16
NW = NC * NS
RPW = BATCH // NW
NREL = 3
NOUT = 8

_SLOT_SRC = [
    (0, 0), (0, 1),
    (1, 0), (1, 1), (1, 2),
    (2, 0), (2, 2),
    (3, 1), (3, 2),
    (4, 0), (4, 1),
    (5, 0), (5, 2),
    (2, 1), (3, 0), (5, 1),
]
_NA = 5
_NB = 8


def _vsqrt(x):
    xc = jnp.maximum(x, jnp.float32(1e-30))
    i = lax.bitcast_convert_type(xc, jnp.int32)
    i = jnp.int32(0x5F3759DF) - jnp.right_shift(i, jnp.int32(1))
    g = lax.bitcast_convert_type(i, jnp.float32)
    for _ in range(3):
        g = g * (jnp.float32(1.5) - jnp.float32(0.5) * xc * g * g)
    return x * g


def _relu(x):
    return jnp.maximum(x, jnp.float32(0.0))


def _sc_body(idx_hbm, cls_hbm, rel_hbm, out_hbm, *refs):
    idx_v, bufa, bufb, bufr, partials, sema, semb = refs

    wid = lax.axis_index("s") * NC + lax.axis_index("c")

    nidx = len(_SLOT_SRC) * RPW
    pltpu.sync_copy(idx_hbm.at[pl.ds(wid * nidx, nidx)], idx_v)

    iota = lax.iota(jnp.int32, L)

    c1 = pltpu.async_copy(cls_hbm.at[idx_v.at[pl.ds(0, _NA * RPW)]], bufa,
                          sema)
    c2 = pltpu.async_copy(cls_hbm.at[idx_v.at[pl.ds(_NA * RPW, _NB * RPW)]],
                          bufb, semb)
    c3 = pltpu.async_copy(
        rel_hbm.at[idx_v.at[pl.ds((_NA + _NB) * RPW, NREL * RPW)]], bufr, semb)

    zero = jnp.zeros((L,), jnp.float32)
    half = jnp.float32(0.5)

    row_a = [iota + jnp.int32(k * RPW) for k in range(_NA)]
    row_b = [iota + jnp.int32(k * RPW) for k in range(_NB)]
    row_r = [iota + jnp.int32(k * RPW) for k in range(NREL)]

    def alo(col, dsp):
        return plsc.load_gather(bufa, [row_a[col], dsp])

    def ahi(col, dsph):
        return jnp.abs(plsc.load_gather(bufa, [row_a[col], dsph]))

    def blo(col, dsp):
        return plsc.load_gather(bufb, [row_b[col], dsp])

    def bhi(col, dsph):
        return jnp.abs(plsc.load_gather(bufb, [row_b[col], dsph]))

    def rlo(k, dsp):
        return plsc.load_gather(bufr, [row_r[k], dsp])

    def rot(dd):
        dsp = jnp.bitwise_and(iota + dd, jnp.int32(DIMH - 1))
        return dsp, dsp + jnp.int32(DIMH)

    c1.wait()

    @plsc.parallel_loop(0, DIMH, step=1, unroll=1, carry=(zero, zero, zero))
    def loop_a(dd, carry):
        s1, ar, br = carry
        dsp, dsph = rot(dd)
        t = _relu(jnp.abs(alo(0, dsp) - alo(1, dsp))
                  + ahi(0, dsph) - ahi(1, dsph))
        s1 = s1 + t * t
        c1v = alo(2, dsp)
        c2v = ahi(2, dsph)
        d1 = alo(3, dsp)
        d2 = ahi(3, dsph)
        e1 = alo(4, dsp)
        e2 = ahi(4, dsph)
        st = jnp.maximum(c1v - c2v, d1 - d2)
        en = jnp.minimum(c1v + c2v, d1 + d2)
        diff = st - en
        ta = _relu(jnp.abs(half * (st + en) - e1) + half * jnp.abs(diff) - e2)
        ar = ar + ta * ta
        tb = _relu(diff)
        br = br + tb * tb
        return s1, ar, br

    s134, a2, b2 = loop_a

    c2.wait()
    c3.wait()

    @plsc.parallel_loop(0, DIMH, step=1, unroll=1, carry=(zero, zero, zero))
    def loop_b(dd, carry):
        s34, djr, negr = carry
        dsp, dsph = rot(dd)
        t = _relu(jnp.abs(blo(0, dsp) + rlo(0, dsp) - blo(1, dsp))
                  + bhi(0, dsph) - bhi(1, dsph))
        s34 = s34 + t * t
        t = _relu(jnp.abs(blo(2, dsp) - rlo(1, dsp) - blo(3, dsp))
                  - bhi(2, dsph) - bhi(3, dsph))
        s34 = s34 + t * t
        t = _relu(jnp.abs(blo(4, dsp) - blo(5, dsp))
                  - bhi(4, dsph) - bhi(5, dsph))
        djr = djr + t * t
        t = _relu(jnp.abs(blo(6, dsp) + rlo(2, dsp) - blo(7, dsp))
                  - bhi(6, dsph) - bhi(7, dsph))
        negr = negr + t * t
        return s34, djr, negr

    s34, djr, negr = loop_b
    s134 = s134 + s34

    two = jnp.float32(2.0)
    djv = _relu(two - _vsqrt(djr))
    negv = two - _vsqrt(negr)

    partials[0, :] = s134
    partials[1, :] = a2
    partials[2, :] = _vsqrt(a2)
    partials[3, :] = b2
    partials[4, :] = _vsqrt(b2)
    partials[5, :] = djv * djv
    partials[6, :] = negv * negv
    partials[7, :] = zero
    pltpu.sync_copy(partials, out_hbm.at[wid])


def _finish_body(x_ref, o_ref):
    x = x_ref[...]
    inv = jnp.float32(1.0 / BATCH)
    s134 = jnp.sum(x[:, 0, :])
    sa2 = jnp.sum(x[:, 1, :])
    sa = jnp.sum(x[:, 2, :])
    sb2 = jnp.sum(x[:, 3, :])
    sb = jnp.sum(x[:, 4, :])
    sdj = jnp.sum(x[:, 5, :])
    sneg = jnp.sum(x[:, 6, :])
    loss2 = inv * sa2 + inv * sb2 + jnp.float32(2.0) * (inv * sa) * (inv * sb)
    total = inv * s134 + loss2 + inv * sdj + inv * sneg
    o_ref[...] = jnp.broadcast_to(total, (1, 1))


@jax.jit
def _run(idx3, classEmb, relEmb):
    mesh = plsc.VectorSubcoreMesh(core_axis_name="c", subcore_axis_name="s")
    scratch = [
        pltpu.VMEM((len(_SLOT_SRC) * RPW,), jnp.int32),
        pltpu.VMEM((_NA * RPW, 2 * DIMH), jnp.float32),
        pltpu.VMEM((_NB * RPW, 2 * DIMH), jnp.float32),
        pltpu.VMEM((NREL * RPW, DIMH), jnp.float32),
        pltpu.VMEM((NOUT, L), jnp.float32),
        pltpu.SemaphoreType.DMA,
        pltpu.SemaphoreType.DMA,
    ]
    sc_call = pl.kernel(
        _sc_body,
        out_type=jax.ShapeDtypeStruct((NW, NOUT, L), jnp.float32),
        mesh=mesh,
        scratch_types=scratch,
        compiler_params=pltpu.CompilerParams(needs_layout_passes=False),
    )
    partials = sc_call(idx3, classEmb, relEmb)
    out = pl.pallas_call(
        _finish_body,
        out_shape=jax.ShapeDtypeStruct((1, 1), jnp.float32),
    )(partials)
    return jnp.reshape(out, ())


def kernel(nf1, nf2, nf3, nf4, disjoint, nf3_neg, classEmb, relEmb):
    b = BATCH
    i32 = jnp.int32
    cols = [
        nf1[:b, 0], nf1[:b, 1],
        nf2[:b, 0], nf2[:b, 1], nf2[:b, 2],
        nf3[:b, 0], nf3[:b, 2],
        nf4[:b, 1], nf4[:b, 2],
        disjoint[:b, 0], disjoint[:b, 1],
        nf3_neg[:b, 0], nf3_neg[:b, 2],
        nf3[:b, 1], nf4[:b, 0], nf3_neg[:b, 1],
    ]
    idx_all = jnp.stack([c.astype(i32) for c in cols], axis=0)
    idx3 = idx_all.reshape(16, NW, RPW).transpose(1, 0, 2).reshape(NW * 256)
    return _run(idx3, classEmb, relEmb)
